# Initial kernel scaffold; baseline (speedup 1.0000x reference)
import jax
import jax.numpy as jnp
from jax.experimental import pallas as pl


def kernel(inputs, probs, training):
    raise NotImplementedError("write your pallas kernel here")


# speedup vs baseline: 16.3097x; 16.3097x over previous
import jax
import jax.numpy as jnp
from jax.experimental import pallas as pl


def _masking_kernel(x_ref, kp1_ref, train_ref, out_ref):
    x = x_ref[...]
    kp1 = kp1_ref[...]

    bits = jax.lax.bitcast_convert_type(x, jnp.int32)
    bu = bits.astype(jnp.uint32)
    keys = jnp.where(bits < 0, ~bu, bu | jnp.uint32(0x80000000))

    p = jnp.zeros((x.shape[0], 1), dtype=jnp.uint32)
    for j in range(31, -1, -1):
        c = p | jnp.uint32(1 << j)
        cnt = jnp.sum((keys < c).astype(jnp.int32), axis=1, keepdims=True)
        p = jnp.where(cnt >= kp1, p, c)

    thr_u = jnp.where(p >= jnp.uint32(0x80000000), p ^ jnp.uint32(0x80000000), ~p)
    thr = jax.lax.bitcast_convert_type(thr_u.astype(jnp.int32), jnp.float32)

    masked = jnp.where(x < thr, 0.0, x)
    num = jnp.sum(x, axis=1, keepdims=True)
    den = jnp.sum(masked, axis=1, keepdims=True)
    scale = jnp.abs(jnp.where(den == 0.0, jnp.zeros_like(num), num / den))
    tb = train_ref[...]
    out_ref[...] = jnp.where(tb != 0, scale * masked, x)


def kernel(inputs, probs, training):
    B, N = inputs.shape
    idx = jnp.maximum(jnp.ceil(jnp.float32(N) * probs).astype(jnp.int32) - 1, 0)
    kp1 = (idx + 1).reshape(B, 1)
    train = jnp.asarray(training, jnp.int32).reshape(1, 1)
    return pl.pallas_call(
        _masking_kernel,
        out_shape=jax.ShapeDtypeStruct((B, N), inputs.dtype),
    )(inputs, kp1, train)
